# baseline (speedup 1.0000x reference)
import functools

import jax
import jax.numpy as jnp
from jax import lax
from jax.experimental import pallas as pl
from jax.experimental.pallas import tpu as pltpu
from jax.experimental.pallas import tpu_sc as plsc

VOCAB = 1000000
DIM = 64
B = 16384
CTX = 20
NEG = 5
W = NEG + 1

L = 16
NC, NS = 2, 16
NW = NC * NS
BPW = B // NW
G = 64
NCH = BPW // G
UROWS = G * CTX
WROWS = G * W
UIB = UROWS // 128
WIB = WROWS // 128
UIPAD = 16
WIPAD = 8
ND = DIM // L

_mesh = plsc.VectorSubcoreMesh(core_axis_name="c", subcore_axis_name="s")


PACK_BLK = 16384
PACK_GRID = -(-VOCAB // PACK_BLK)


def _tc_pack_body(ut_ref, wt_ref, out_ref):
    eye = (lax.broadcasted_iota(jnp.int32, (DIM, DIM), 0)
           == lax.broadcasted_iota(jnp.int32, (DIM, DIM), 1)
           ).astype(jnp.float32)
    dn = (((0,), (0,)), ((), ()))
    tu = lax.dot_general(ut_ref[...], eye, dn)
    tw = lax.dot_general(wt_ref[...], eye, dn)
    out_ref[...] = jnp.concatenate([tu, tw], axis=1)


@functools.partial(
    pl.kernel,
    out_type=(
        jax.ShapeDtypeStruct((B // 2, 128), jnp.float32),
        jax.ShapeDtypeStruct((B // 2, 128), jnp.float32),
        jax.ShapeDtypeStruct((B // 2, 128), jnp.float32),
    ),
    mesh=_mesh,
    compiler_params=pltpu.CompilerParams(use_tc_tiling_on_sc=False),
    scratch_types=[
        pltpu.VMEM((UIPAD, 128), jnp.int32),
        pltpu.VMEM((WIPAD, 128), jnp.int32),
        pltpu.VMEM((UROWS, DIM), jnp.float32),
        pltpu.VMEM((WROWS, DIM), jnp.float32),
        pltpu.VMEM((G // 2, 128), jnp.float32),
        pltpu.VMEM((G // 2, 128), jnp.float32),
        pltpu.VMEM((G // 2, 128), jnp.float32),
        pltpu.SemaphoreType.DMA,
    ],
)
def _sc_gather_pool(uidx_hbm, widx_hbm, comb, su_hbm, sw_hbm, sn_hbm,
                    uidx_v, widx_v, urows_v, wrows_v, su_v, sw_v, sn_v, gsem):
    wid = lax.axis_index("s") * NC + lax.axis_index("c")
    ebase = wid * BPW

    def chunk(g, carry):
        e0 = ebase + g * G
        q = wid * NCH + g
        pltpu.sync_copy(uidx_hbm.at[q], uidx_v)
        pltpu.sync_copy(widx_hbm.at[q], widx_v)
        for j in range(UIB):
            pltpu.async_copy(comb.at[uidx_v.at[j]],
                             urows_v.at[pl.ds(j * 128, 128)], gsem)
        for j in range(WIB):
            pltpu.async_copy(comb.at[widx_v.at[j]],
                             wrows_v.at[pl.ds(j * 128, 128)], gsem)
        for j in range(UIB):
            pltpu.make_async_copy(comb.at[uidx_v.at[j]],
                                  urows_v.at[pl.ds(j * 128, 128)], gsem).wait()
        for j in range(WIB):
            pltpu.make_async_copy(comb.at[widx_v.at[j]],
                                  wrows_v.at[pl.ds(j * 128, 128)], gsem).wait()

        def per_ex(e, carry2):
            zero = jnp.zeros((L,), jnp.float32)
            half = (e % 2) * DIM

            def uacc(c, accs):
                r = e * CTX + c
                return tuple(accs[d] + urows_v[r, pl.ds(d * L, L)]
                             for d in range(ND))

            su = lax.fori_loop(0, CTX, uacc, (zero,) * ND)

            def nacc(n, accs):
                r = e * W + 1 + n
                return tuple(accs[d] + wrows_v[r, pl.ds(d * L, L)]
                             for d in range(ND))

            sn = lax.fori_loop(0, NEG, nacc, (zero,) * ND)

            for d in range(ND):
                su_v[e // 2, pl.ds(half + d * L, L)] = su[d]
                sw_v[e // 2, pl.ds(half + d * L, L)] = (
                    wrows_v[e * W, pl.ds(d * L, L)])
                sn_v[e // 2, pl.ds(half + d * L, L)] = sn[d]
            return carry2

        lax.fori_loop(0, G, per_ex, 0)

        pltpu.sync_copy(su_v, su_hbm.at[pl.ds(e0 // 2, G // 2)])
        pltpu.sync_copy(sw_v, sw_hbm.at[pl.ds(e0 // 2, G // 2)])
        pltpu.sync_copy(sn_v, sn_hbm.at[pl.ds(e0 // 2, G // 2)])
        return carry

    lax.fori_loop(0, NCH, chunk, 0)


TC_BLK = 1024


def _logsig(x):
    return jnp.minimum(x, 0.0) - jnp.log1p(jnp.exp(-jnp.abs(x)))


def _tc_score_body(su_ref, sw_ref, sn_ref, out_ref):
    i = pl.program_id(0)
    su = su_ref[...]
    sw = sw_ref[...]
    sn = sn_ref[...]
    p = su * sw
    q = su * sn
    s2a = jnp.sum(p[:, :DIM], axis=1)
    s2b = jnp.sum(p[:, DIM:], axis=1)
    n2a = jnp.sum(q[:, :DIM], axis=1)
    n2b = jnp.sum(q[:, DIM:], axis=1)
    part = (jnp.sum(_logsig(s2a)) + jnp.sum(_logsig(s2b))
            + jnp.sum(_logsig(-n2a)) + jnp.sum(_logsig(-n2b)))

    @pl.when(i == 0)
    def _init():
        out_ref[...] = jnp.zeros_like(out_ref)

    out_ref[...] += part


def kernel(pos_u, pos_w, neg_w, u_table, w_table):
    uidx = jnp.pad(
        (2 * pos_u.astype(jnp.int32)).reshape(B // G, UIB, 128),
        ((0, 0), (0, UIPAD - UIB), (0, 0)))
    widx = jnp.pad(
        (2 * jnp.concatenate(
            [pos_w.astype(jnp.int32)[:, None], neg_w.astype(jnp.int32)],
            axis=1) + 1).reshape(B // G, WIB, 128),
        ((0, 0), (0, WIPAD - WIB), (0, 0)))

    comb = pl.pallas_call(
        _tc_pack_body,
        grid=(PACK_GRID,),
        in_specs=[pl.BlockSpec((DIM, PACK_BLK), lambda i: (0, i))] * 2,
        out_specs=pl.BlockSpec((PACK_BLK, 128), lambda i: (i, 0)),
        out_shape=jax.ShapeDtypeStruct((VOCAB, 128), jnp.float32),
    )(u_table.T, w_table.T)

    su2, sw2, sn2 = _sc_gather_pool(uidx, widx, comb.reshape(2 * VOCAB, DIM))

    loss = pl.pallas_call(
        _tc_score_body,
        grid=(B // 2 // TC_BLK,),
        in_specs=[pl.BlockSpec((TC_BLK, 128), lambda i: (i, 0))] * 3,
        out_specs=pl.BlockSpec((1, 1), lambda i: (0, 0)),
        out_shape=jax.ShapeDtypeStruct((1, 1), jnp.float32),
    )(su2, sw2, sn2)
    return -loss[0, 0]


# speedup vs baseline: 1.0187x; 1.0187x over previous
import functools

import jax
import jax.numpy as jnp
from jax import lax
from jax.experimental import pallas as pl
from jax.experimental.pallas import tpu as pltpu
from jax.experimental.pallas import tpu_sc as plsc

VOCAB = 1000000
DIM = 64
B = 16384
CTX = 20
NEG = 5
W = NEG + 1

L = 16
NC, NS = 2, 16
NW = NC * NS
BPW = B // NW
ND = DIM // L

GU = 64
NCHU = BPW // GU
UROWS = GU * CTX
UIB = UROWS // 128
UIPAD = 16

GW = 128
NCHW = BPW // GW
WROWS = GW * W
WIB = WROWS // 128
WIPAD = 8

_mesh = plsc.VectorSubcoreMesh(core_axis_name="c", subcore_axis_name="s")


SPLIT = 524288
PB = 8192
PACK_GRID = SPLIT // PB


def _tc_pack_body(lo_ref, hi_ref, out_ref):
    out_ref[...] = jnp.concatenate([lo_ref[...].T, hi_ref[...].T], axis=1)


def _pack(table_t):
    return pl.pallas_call(
        _tc_pack_body,
        grid=(PACK_GRID,),
        in_specs=[
            pl.BlockSpec((DIM, PB), lambda i: (0, i)),
            pl.BlockSpec(
                (DIM, PB),
                lambda i: (0, jnp.minimum(i + PACK_GRID, VOCAB // PB))),
        ],
        out_specs=pl.BlockSpec((PB, 128), lambda i: (i, 0)),
        out_shape=jax.ShapeDtypeStruct((SPLIT, 128), jnp.float32),
    )(table_t, table_t).reshape(2 * SPLIT, DIM)


def _remap(idx):
    i = idx.astype(jnp.int32)
    return jnp.where(i < SPLIT, 2 * i, 2 * (i - SPLIT) + 1)


@functools.partial(
    pl.kernel,
    out_type=jax.ShapeDtypeStruct((B // 2, 128), jnp.float32),
    mesh=_mesh,
    compiler_params=pltpu.CompilerParams(use_tc_tiling_on_sc=False),
    scratch_types=[
        pltpu.VMEM((UIPAD, 128), jnp.int32),
        pltpu.VMEM((UROWS, DIM), jnp.float32),
        pltpu.VMEM((GU // 2, 128), jnp.float32),
        pltpu.SemaphoreType.DMA,
    ],
)
def _sc_gather_u(uidx_hbm, u2, su_hbm, uidx_v, urows_v, su_v, gsem):
    wid = lax.axis_index("s") * NC + lax.axis_index("c")
    ebase = wid * BPW

    def chunk(g, carry):
        e0 = ebase + g * GU
        q = wid * NCHU + g
        pltpu.sync_copy(uidx_hbm.at[q], uidx_v)
        for j in range(UIB):
            pltpu.async_copy(u2.at[uidx_v.at[j]],
                             urows_v.at[pl.ds(j * 128, 128)], gsem)
        for j in range(UIB):
            pltpu.make_async_copy(u2.at[uidx_v.at[j]],
                                  urows_v.at[pl.ds(j * 128, 128)], gsem).wait()

        def per_ex(e, carry2):
            zero = jnp.zeros((L,), jnp.float32)
            half = (e % 2) * DIM

            def uacc(c, accs):
                r = e * CTX + c
                return tuple(accs[d] + urows_v[r, pl.ds(d * L, L)]
                             for d in range(ND))

            su = lax.fori_loop(0, CTX, uacc, (zero,) * ND)
            for d in range(ND):
                su_v[e // 2, pl.ds(half + d * L, L)] = su[d]
            return carry2

        lax.fori_loop(0, GU, per_ex, 0)
        pltpu.sync_copy(su_v, su_hbm.at[pl.ds(e0 // 2, GU // 2)])
        return carry

    lax.fori_loop(0, NCHU, chunk, 0)


@functools.partial(
    pl.kernel,
    out_type=(
        jax.ShapeDtypeStruct((B // 2, 128), jnp.float32),
        jax.ShapeDtypeStruct((B // 2, 128), jnp.float32),
    ),
    mesh=_mesh,
    compiler_params=pltpu.CompilerParams(use_tc_tiling_on_sc=False),
    scratch_types=[
        pltpu.VMEM((WIPAD, 128), jnp.int32),
        pltpu.VMEM((WROWS, DIM), jnp.float32),
        pltpu.VMEM((GW // 2, 128), jnp.float32),
        pltpu.VMEM((GW // 2, 128), jnp.float32),
        pltpu.SemaphoreType.DMA,
    ],
)
def _sc_gather_w(widx_hbm, w2, sw_hbm, sn_hbm,
                 widx_v, wrows_v, sw_v, sn_v, gsem):
    wid = lax.axis_index("s") * NC + lax.axis_index("c")
    ebase = wid * BPW

    def chunk(g, carry):
        e0 = ebase + g * GW
        q = wid * NCHW + g
        pltpu.sync_copy(widx_hbm.at[q], widx_v)
        for j in range(WIB):
            pltpu.async_copy(w2.at[widx_v.at[j]],
                             wrows_v.at[pl.ds(j * 128, 128)], gsem)
        for j in range(WIB):
            pltpu.make_async_copy(w2.at[widx_v.at[j]],
                                  wrows_v.at[pl.ds(j * 128, 128)], gsem).wait()

        def per_ex(e, carry2):
            zero = jnp.zeros((L,), jnp.float32)
            half = (e % 2) * DIM

            def nacc(n, accs):
                r = e * W + 1 + n
                return tuple(accs[d] + wrows_v[r, pl.ds(d * L, L)]
                             for d in range(ND))

            sn = lax.fori_loop(0, NEG, nacc, (zero,) * ND)
            for d in range(ND):
                sw_v[e // 2, pl.ds(half + d * L, L)] = (
                    wrows_v[e * W, pl.ds(d * L, L)])
                sn_v[e // 2, pl.ds(half + d * L, L)] = sn[d]
            return carry2

        lax.fori_loop(0, GW, per_ex, 0)
        pltpu.sync_copy(sw_v, sw_hbm.at[pl.ds(e0 // 2, GW // 2)])
        pltpu.sync_copy(sn_v, sn_hbm.at[pl.ds(e0 // 2, GW // 2)])
        return carry

    lax.fori_loop(0, NCHW, chunk, 0)


TC_BLK = 1024


def _logsig(x):
    return jnp.minimum(x, 0.0) - jnp.log1p(jnp.exp(-jnp.abs(x)))


def _tc_score_body(su_ref, sw_ref, sn_ref, out_ref):
    i = pl.program_id(0)
    su = su_ref[...]
    sw = sw_ref[...]
    sn = sn_ref[...]
    p = su * sw
    q = su * sn
    s2a = jnp.sum(p[:, :DIM], axis=1)
    s2b = jnp.sum(p[:, DIM:], axis=1)
    n2a = jnp.sum(q[:, :DIM], axis=1)
    n2b = jnp.sum(q[:, DIM:], axis=1)
    part = (jnp.sum(_logsig(s2a)) + jnp.sum(_logsig(s2b))
            + jnp.sum(_logsig(-n2a)) + jnp.sum(_logsig(-n2b)))

    @pl.when(i == 0)
    def _init():
        out_ref[...] = jnp.zeros_like(out_ref)

    out_ref[...] += part


def kernel(pos_u, pos_w, neg_w, u_table, w_table):
    uidx = jnp.pad(
        _remap(pos_u).reshape(B // GU, UIB, 128),
        ((0, 0), (0, UIPAD - UIB), (0, 0)))
    widx = jnp.pad(
        _remap(jnp.concatenate(
            [pos_w[:, None], neg_w], axis=1)).reshape(B // GW, WIB, 128),
        ((0, 0), (0, WIPAD - WIB), (0, 0)))

    u2 = _pack(u_table.T)
    su2 = _sc_gather_u(uidx, u2)
    w2 = _pack(w_table.T)
    sw2, sn2 = _sc_gather_w(widx, w2)

    loss = pl.pallas_call(
        _tc_score_body,
        grid=(B // 2 // TC_BLK,),
        in_specs=[pl.BlockSpec((TC_BLK, 128), lambda i: (i, 0))] * 3,
        out_specs=pl.BlockSpec((1, 1), lambda i: (0, 0)),
        out_shape=jax.ShapeDtypeStruct((1, 1), jnp.float32),
    )(su2, sw2, sn2)
    return -loss[0, 0]


# speedup vs baseline: 1.0693x; 1.0497x over previous
import functools

import jax
import jax.numpy as jnp
from jax import lax
from jax.experimental import pallas as pl
from jax.experimental.pallas import tpu as pltpu
from jax.experimental.pallas import tpu_sc as plsc

VOCAB = 1000000
DIM = 64
B = 16384
CTX = 20
NEG = 5
W = NEG + 1

L = 16
NC, NS = 2, 16
NW = NC * NS
BPW = B // NW
ND = DIM // L

GU = 64
NCHU = BPW // GU
UROWS = GU * CTX
UIB = UROWS // 128
UIPAD = 16

GW = 128
NCHW = BPW // GW
WROWS = GW * W
WIB = WROWS // 128
WIPAD = 8

_mesh = plsc.VectorSubcoreMesh(core_axis_name="c", subcore_axis_name="s")


SPLIT = 524288
PB = 16384
PACK_GRID = SPLIT // PB


def _tc_pack_body(lo_ref, hi_ref, out_ref):
    out_ref[...] = jnp.concatenate([lo_ref[...].T, hi_ref[...].T], axis=1)


def _pack(table_t):
    return pl.pallas_call(
        _tc_pack_body,
        grid=(PACK_GRID,),
        in_specs=[
            pl.BlockSpec((DIM, PB), lambda i: (0, i)),
            pl.BlockSpec(
                (DIM, PB),
                lambda i: (0, jnp.minimum(i + PACK_GRID, VOCAB // PB))),
        ],
        out_specs=pl.BlockSpec((PB, 128), lambda i: (i, 0)),
        out_shape=jax.ShapeDtypeStruct((SPLIT, 128), jnp.float32),
    )(table_t, table_t).reshape(2 * SPLIT, DIM)


def _remap(idx):
    i = idx.astype(jnp.int32)
    return jnp.where(i < SPLIT, 2 * i, 2 * (i - SPLIT) + 1)


@functools.partial(
    pl.kernel,
    out_type=jax.ShapeDtypeStruct((B // 2, 128), jnp.float32),
    mesh=_mesh,
    compiler_params=pltpu.CompilerParams(use_tc_tiling_on_sc=False),
    scratch_types=[
        pltpu.VMEM((UIPAD, 128), jnp.int32),
        pltpu.VMEM((UROWS, DIM), jnp.float32),
        pltpu.VMEM((GU // 2, 128), jnp.float32),
        pltpu.SemaphoreType.DMA,
    ],
)
def _sc_gather_u(uidx_hbm, u2, su_hbm, uidx_v, urows_v, su_v, gsem):
    wid = lax.axis_index("s") * NC + lax.axis_index("c")
    ebase = wid * BPW

    def chunk(g, carry):
        e0 = ebase + g * GU
        q = wid * NCHU + g
        pltpu.sync_copy(uidx_hbm.at[q], uidx_v)
        for j in range(UIB):
            pltpu.async_copy(u2.at[uidx_v.at[j]],
                             urows_v.at[pl.ds(j * 128, 128)], gsem)
        for j in range(UIB):
            pltpu.make_async_copy(u2.at[uidx_v.at[j]],
                                  urows_v.at[pl.ds(j * 128, 128)], gsem).wait()

        def per_ex(e, carry2):
            zero = jnp.zeros((L,), jnp.float32)
            half = (e % 2) * DIM

            def uacc(c, accs):
                r = e * CTX + c
                return tuple(accs[d] + urows_v[r, pl.ds(d * L, L)]
                             for d in range(ND))

            su = lax.fori_loop(0, CTX, uacc, (zero,) * ND)
            for d in range(ND):
                su_v[e // 2, pl.ds(half + d * L, L)] = su[d]
            return carry2

        lax.fori_loop(0, GU, per_ex, 0)
        pltpu.sync_copy(su_v, su_hbm.at[pl.ds(e0 // 2, GU // 2)])
        return carry

    lax.fori_loop(0, NCHU, chunk, 0)


@functools.partial(
    pl.kernel,
    out_type=(
        jax.ShapeDtypeStruct((B // 2, 128), jnp.float32),
        jax.ShapeDtypeStruct((B // 2, 128), jnp.float32),
    ),
    mesh=_mesh,
    compiler_params=pltpu.CompilerParams(use_tc_tiling_on_sc=False),
    scratch_types=[
        pltpu.VMEM((WIPAD, 128), jnp.int32),
        pltpu.VMEM((WROWS, DIM), jnp.float32),
        pltpu.VMEM((GW // 2, 128), jnp.float32),
        pltpu.VMEM((GW // 2, 128), jnp.float32),
        pltpu.SemaphoreType.DMA,
    ],
)
def _sc_gather_w(widx_hbm, w2, sw_hbm, sn_hbm,
                 widx_v, wrows_v, sw_v, sn_v, gsem):
    wid = lax.axis_index("s") * NC + lax.axis_index("c")
    ebase = wid * BPW

    def chunk(g, carry):
        e0 = ebase + g * GW
        q = wid * NCHW + g
        pltpu.sync_copy(widx_hbm.at[q], widx_v)
        for j in range(WIB):
            pltpu.async_copy(w2.at[widx_v.at[j]],
                             wrows_v.at[pl.ds(j * 128, 128)], gsem)
        for j in range(WIB):
            pltpu.make_async_copy(w2.at[widx_v.at[j]],
                                  wrows_v.at[pl.ds(j * 128, 128)], gsem).wait()

        def per_ex(e, carry2):
            zero = jnp.zeros((L,), jnp.float32)
            half = (e % 2) * DIM

            def nacc(n, accs):
                r = e * W + 1 + n
                return tuple(accs[d] + wrows_v[r, pl.ds(d * L, L)]
                             for d in range(ND))

            sn = lax.fori_loop(0, NEG, nacc, (zero,) * ND)
            for d in range(ND):
                sw_v[e // 2, pl.ds(half + d * L, L)] = (
                    wrows_v[e * W, pl.ds(d * L, L)])
                sn_v[e // 2, pl.ds(half + d * L, L)] = sn[d]
            return carry2

        lax.fori_loop(0, GW, per_ex, 0)
        pltpu.sync_copy(sw_v, sw_hbm.at[pl.ds(e0 // 2, GW // 2)])
        pltpu.sync_copy(sn_v, sn_hbm.at[pl.ds(e0 // 2, GW // 2)])
        return carry

    lax.fori_loop(0, NCHW, chunk, 0)


TC_BLK = 1024


def _logsig(x):
    return jnp.minimum(x, 0.0) - jnp.log1p(jnp.exp(-jnp.abs(x)))


def _tc_score_body(su_ref, sw_ref, sn_ref, out_ref):
    i = pl.program_id(0)
    su = su_ref[...]
    sw = sw_ref[...]
    sn = sn_ref[...]
    p = su * sw
    q = su * sn
    s2a = jnp.sum(p[:, :DIM], axis=1)
    s2b = jnp.sum(p[:, DIM:], axis=1)
    n2a = jnp.sum(q[:, :DIM], axis=1)
    n2b = jnp.sum(q[:, DIM:], axis=1)
    part = (jnp.sum(_logsig(s2a)) + jnp.sum(_logsig(s2b))
            + jnp.sum(_logsig(-n2a)) + jnp.sum(_logsig(-n2b)))

    @pl.when(i == 0)
    def _init():
        out_ref[...] = jnp.zeros_like(out_ref)

    out_ref[...] += part


def kernel(pos_u, pos_w, neg_w, u_table, w_table):
    uidx = jnp.pad(
        _remap(pos_u).reshape(B // GU, UIB, 128),
        ((0, 0), (0, UIPAD - UIB), (0, 0)))
    widx = jnp.pad(
        _remap(jnp.concatenate(
            [pos_w[:, None], neg_w], axis=1)).reshape(B // GW, WIB, 128),
        ((0, 0), (0, WIPAD - WIB), (0, 0)))

    u2 = _pack(u_table.T)
    su2 = _sc_gather_u(uidx, u2)
    w2 = _pack(w_table.T)
    sw2, sn2 = _sc_gather_w(widx, w2)

    loss = pl.pallas_call(
        _tc_score_body,
        grid=(B // 2 // TC_BLK,),
        in_specs=[pl.BlockSpec((TC_BLK, 128), lambda i: (i, 0))] * 3,
        out_specs=pl.BlockSpec((1, 1), lambda i: (0, 0)),
        out_shape=jax.ShapeDtypeStruct((1, 1), jnp.float32),
    )(su2, sw2, sn2)
    return -loss[0, 0]
